# baseline (speedup 1.0000x reference)
import functools

import jax
import jax.numpy as jnp
from jax import lax
from jax.experimental import pallas as pl
from jax.experimental.pallas import tpu as pltpu
from jax.experimental.pallas import tpu_sc as plsc

BATCH = 4096
SEQ = 50
EMBED = 32
N = BATCH * SEQ
CHUNK = 128
NROWS = N // CHUNK
K = 10
_info = plsc.get_sparse_core_info()
NC, NS = _info.num_cores, _info.num_subcores
NW = NC * NS
RPW = NROWS // NW
G = RPW // K


def _body(idx_hbm, table_hbm, out_hbm, idx_v, rows_v, sem):
    w = lax.axis_index("s") * NC + lax.axis_index("c")
    pltpu.sync_copy(idx_hbm.at[w], idx_v)

    @pl.loop(0, G)
    def _group(g):
        descs = [
            pltpu.async_copy(
                table_hbm.at[idx_v.at[g * K + j]],
                rows_v.at[pl.ds(j * CHUNK, CHUNK)],
                sem,
            )
            for j in range(K)
        ]
        for d in descs:
            d.wait()
        pltpu.sync_copy(rows_v, out_hbm.at[w, pl.ds(g * K * CHUNK, K * CHUNK)])


@jax.jit
def _gather(idx3d, table):
    mesh = plsc.VectorSubcoreMesh(core_axis_name="c", subcore_axis_name="s")
    f = pl.kernel(
        _body,
        out_type=jax.ShapeDtypeStruct((NW, RPW * CHUNK, EMBED), jnp.float32),
        mesh=mesh,
        scratch_types=[
            pltpu.VMEM((RPW, CHUNK), jnp.int32),
            pltpu.VMEM((K * CHUNK, EMBED), jnp.float32),
            pltpu.SemaphoreType.DMA,
        ],
        compiler_params=pltpu.CompilerParams(use_tc_tiling_on_sc=False),
    )
    return f(idx3d, table)


def kernel(scentences, table):
    idx3d = scentences.astype(jnp.int32).reshape(NW, RPW, CHUNK)
    out = _gather(idx3d, table)
    return out.reshape(BATCH, SEQ, EMBED)
